# Initial kernel scaffold; baseline (speedup 1.0000x reference)
import jax
import jax.numpy as jnp
from jax.experimental import pallas as pl


def kernel(z, W):
    raise NotImplementedError("write your pallas kernel here")


# speedup vs baseline: 1.1266x; 1.1266x over previous
import functools

import jax
import jax.numpy as jnp
from jax import lax
from jax.experimental import pallas as pl
from jax.experimental.pallas import tpu as pltpu
from jax.experimental.pallas import tpu_sc as plsc

N_BLK = 1024
K_BLK = 2048


def _argmin_body(x_ref, w_ref, a_ref, b_ref, ind_ref, min_s, ind_s):
    j = pl.program_id(1)
    nk = pl.num_programs(1)

    @pl.when(j == 0)
    def _init():
        min_s[...] = jnp.full_like(min_s[...], jnp.inf)
        ind_s[...] = jnp.zeros_like(ind_s[...])

    xb = (2.0 * x_ref[...]).astype(jnp.bfloat16)
    g = lax.dot_general(xb, w_ref[...],
                        dimension_numbers=(((1,), (1,)), ((), ())),
                        preferred_element_type=jnp.float32)
    dist = (a_ref[...] - g) + b_ref[...]
    lmin = jnp.min(dist, axis=1, keepdims=True)
    col = lax.broadcasted_iota(jnp.int32, dist.shape, 1) + j * K_BLK
    lind = jnp.min(jnp.where(dist == lmin, col, jnp.int32(2**30)),
                   axis=1, keepdims=True)
    better = lmin < min_s[...]
    ind_s[...] = jnp.where(better, lind, ind_s[...])
    min_s[...] = jnp.where(better, lmin, min_s[...])

    @pl.when(j == nk - 1)
    def _emit():
        ind_ref[...] = ind_s[...]


def _argmin_call(flat, w, a, b):
    n, d = flat.shape
    k = w.shape[0]
    return pl.pallas_call(
        _argmin_body,
        grid=(n // N_BLK, k // K_BLK),
        in_specs=[
            pl.BlockSpec((N_BLK, d), lambda i, j: (i, 0)),
            pl.BlockSpec((K_BLK, d), lambda i, j: (j, 0)),
            pl.BlockSpec((N_BLK, 1), lambda i, j: (i, 0)),
            pl.BlockSpec((1, K_BLK), lambda i, j: (0, j)),
        ],
        out_specs=pl.BlockSpec((N_BLK, 1), lambda i, j: (i, 0)),
        out_shape=jax.ShapeDtypeStruct((n, 1), jnp.int32),
        scratch_shapes=[
            pltpu.VMEM((N_BLK, 1), jnp.float32),
            pltpu.VMEM((N_BLK, 1), jnp.int32),
        ],
    )(flat, w, a, b)


def _sc_gather(table, idx):
    n = idx.shape[0]
    k, d = table.shape
    info = plsc.get_sparse_core_info()
    nw = info.num_cores * info.num_subcores
    bpw = n // nw
    ch = 128
    nch = bpw // ch
    idx3 = idx.reshape(nw, nch, ch)
    mesh = plsc.VectorSubcoreMesh(core_axis_name="c", subcore_axis_name="s")

    @functools.partial(
        pl.kernel,
        out_type=jax.ShapeDtypeStruct((n, d), jnp.float32),
        mesh=mesh,
        scratch_types=[
            pltpu.VMEM((nch, ch), jnp.int32),
            pltpu.VMEM((bpw, d), jnp.float32),
            pltpu.SemaphoreType.DMA,
        ],
        compiler_params=pltpu.CompilerParams(use_tc_tiling_on_sc=False),
    )
    def gather_kernel(table_hbm, idx_hbm, out_hbm, idx_v, rows_v, sem):
        wid = lax.axis_index("s") * info.num_cores + lax.axis_index("c")
        pltpu.sync_copy(idx_hbm.at[wid], idx_v)
        copies = []
        for c in range(nch):
            copies.append(pltpu.async_copy(
                table_hbm.at[idx_v.at[c]], rows_v.at[pl.ds(c * ch, ch)], sem))
        for cp in copies:
            cp.wait()
        pltpu.sync_copy(rows_v, out_hbm.at[pl.ds(wid * bpw, bpw)])

    return gather_kernel(table, idx3)


def _ew_body(z_ref, zq_ref, zo_ref, df_ref):
    zv = z_ref[...]
    qv = zq_ref[...]
    dlt = qv - zv
    zo_ref[...] = zv + dlt
    sq = dlt * dlt
    df_ref[...] = 0.25 * (sq + sq)


def _ew_call(zflat, zqflat):
    rows, cols = zflat.shape
    blk = rows // 4
    return pl.pallas_call(
        _ew_body,
        grid=(rows // blk,),
        in_specs=[
            pl.BlockSpec((blk, cols), lambda i: (i, 0)),
            pl.BlockSpec((blk, cols), lambda i: (i, 0)),
        ],
        out_specs=[
            pl.BlockSpec((blk, cols), lambda i: (i, 0)),
            pl.BlockSpec((blk, cols), lambda i: (i, 0)),
        ],
        out_shape=[
            jax.ShapeDtypeStruct((rows, cols), jnp.float32),
            jax.ShapeDtypeStruct((rows, cols), jnp.float32),
        ],
    )(zflat, zqflat)


def kernel(z, W):
    bsz, s, hid = z.shape
    flat = z.reshape(-1, hid)
    a = jnp.sum(z ** 2, axis=2).reshape(-1, 1)
    b = jnp.sum(W ** 2, axis=1, keepdims=True).T
    ind2 = _argmin_call(flat, W, a, b)
    ind = ind2.reshape(-1)
    zq = _sc_gather(W, ind)
    n = flat.shape[0]
    cols = 1024
    zo, df = _ew_call(flat.reshape(n * hid // cols, cols),
                      zq.reshape(n * hid // cols, cols))
    return (zo.reshape(bsz, s, hid), df.reshape(bsz, s, hid),
            ind.reshape(bsz, s))
